# baseline (speedup 1.0000x reference)
import functools

import jax
import jax.numpy as jnp
from jax import lax
from jax.experimental import pallas as pl
from jax.experimental.pallas import tpu as pltpu
from jax.experimental.pallas import tpu_sc as plsc

MAX_SIZE = 1000000
D = 64
SAMPLE_BS = 4096
LANES = 16


def _make_sc_kernel(data_len: int, num_workers: int):
    chunk = SAMPLE_BS // num_workers
    half = chunk // 2
    mesh = plsc.VectorSubcoreMesh(core_axis_name="c", subcore_axis_name="s")
    num_cores = mesh.num_cores

    @functools.partial(
        pl.kernel,
        mesh=mesh,
        out_type=jax.ShapeDtypeStruct((SAMPLE_BS, D), jnp.float32),
        compiler_params=pltpu.CompilerParams(use_tc_tiling_on_sc=True,
                                             needs_layout_passes=False),
        scratch_types=[
            pltpu.VMEM((chunk,), jnp.int32),
            pltpu.VMEM((chunk,), jnp.int32),
            pltpu.VMEM((chunk,), jnp.int32),
            pltpu.VMEM((LANES,), jnp.int32),
            pltpu.VMEM((half, 8, D), jnp.float32),
            pltpu.VMEM((chunk, D), jnp.float32),
            pltpu.SemaphoreType.DMA,
        ],
    )
    def sc_kernel(mem_hbm, flat_hbm, ip_hbm, sidx_hbm, out_hbm,
                  msk_v, blk_v, sub_v, ip_v, slabs_v, orows_v, sem0):
        wid = lax.axis_index("s") * num_cores + lax.axis_index("c")
        base = pl.multiple_of(wid * chunk, chunk)

        pltpu.sync_copy(sidx_hbm.at[pl.ds(base, chunk)], msk_v)
        pltpu.sync_copy(ip_hbm, ip_v)
        ip = ip_v[...]

        for c in range(chunk // LANES):
            sl = pl.ds(c * LANES, LANES)
            iv = msk_v[sl]
            d = iv - ip
            rel = d + jnp.where(d < 0, MAX_SIZE, 0)
            in_win = rel < data_len
            srow = jnp.where(in_win, rel, iv)
            blk_v[sl] = (srow >> 3) << 3
            sub_v[sl] = srow & 7
            msk_v[sl] = jnp.where(in_win, 1, 0)

        for b in range(2):
            for c16 in range(half // LANES):
                sl = pl.ds(b * half + c16 * LANES, LANES)
                bv = blk_v[sl]
                mv = msk_v[sl]
                for lane in range(LANES):
                    i = c16 * LANES + lane
                    blk = pl.multiple_of(bv[lane], 8)
                    m = mv[lane]

                    @pl.when(m == 1)
                    def _(blk=blk, i=i):
                        b_out = blk >> 7
                        t8 = pl.multiple_of(blk & 127, 8)
                        pltpu.async_copy(flat_hbm.at[b_out].at[pl.ds(t8, 8)],
                                         slabs_v.at[i], sem0)

                    @pl.when(m == 0)
                    def _(blk=blk, i=i):
                        pltpu.async_copy(mem_hbm.at[pl.ds(blk, 8)],
                                         slabs_v.at[i], sem0)

            for i in range(half):
                pltpu.make_async_copy(mem_hbm.at[pl.ds(0, 8)],
                                      slabs_v.at[i], sem0).wait()

            for c16 in range(half // LANES):
                sl = pl.ds(b * half + c16 * LANES, LANES)
                sv = sub_v[sl]
                for lane in range(LANES):
                    i = c16 * LANES + lane
                    s_sub = sv[lane]
                    for k in range(D // LANES):
                        ksl = pl.ds(k * LANES, LANES)
                        orows_v[b * half + i, ksl] = slabs_v[i, s_sub, ksl]

        pltpu.sync_copy(orows_v, out_hbm.at[pl.ds(base, chunk)])

    return sc_kernel


def kernel(mem, transition, insert_position, sample_idx):
    data_len = transition.shape[0] * transition.shape[1]
    info = plsc.get_sparse_core_info()
    num_workers = info.num_cores * info.num_subcores
    ip_arr = jnp.full((LANES,), insert_position, dtype=jnp.int32)
    sc = _make_sc_kernel(data_len, num_workers)
    return sc(mem, transition, ip_arr, sample_idx.astype(jnp.int32))


# speedup vs baseline: 3.7577x; 3.7577x over previous
import functools

import jax
import jax.numpy as jnp
from jax import lax
from jax.experimental import pallas as pl
from jax.experimental.pallas import tpu as pltpu
from jax.experimental.pallas import tpu_sc as plsc

MAX_SIZE = 1000000
D = 64
SAMPLE_BS = 4096
LANES = 16
CB = 128
BATCH = 4


def _make_sc_kernel(data_len: int, num_workers: int):
    chunk = SAMPLE_BS // num_workers
    mesh = plsc.VectorSubcoreMesh(core_axis_name="c", subcore_axis_name="s")
    num_cores = mesh.num_cores

    @functools.partial(
        pl.kernel,
        mesh=mesh,
        out_type=jax.ShapeDtypeStruct((D, SAMPLE_BS), jnp.float32),
        compiler_params=pltpu.CompilerParams(needs_layout_passes=False),
        scratch_types=[
            pltpu.VMEM((chunk,), jnp.int32),
            pltpu.VMEM((chunk,), jnp.int32),
            pltpu.VMEM((chunk,), jnp.int32),
            pltpu.VMEM((LANES,), jnp.int32),
            pltpu.VMEM((BATCH, D, CB), jnp.float32),
            pltpu.VMEM((D, chunk), jnp.float32),
            pltpu.SemaphoreType.DMA,
        ],
    )
    def sc_kernel(memt_hbm, trt_hbm, ip_hbm, sidx_hbm, out_hbm,
                  msk_v, bl_v, t_v, ip_v, slabs_v, out_v, sem0):
        wid = lax.axis_index("s") * num_cores + lax.axis_index("c")
        base = pl.multiple_of(wid * chunk, chunk)

        pltpu.sync_copy(sidx_hbm.at[pl.ds(base, chunk)], msk_v)
        pltpu.sync_copy(ip_hbm, ip_v)
        ip = ip_v[...]

        for c in range(chunk // LANES):
            sl = pl.ds(c * LANES, LANES)
            iv = msk_v[sl]
            d = iv - ip
            rel = d + jnp.where(d < 0, MAX_SIZE, 0)
            in_win = rel < data_len
            srow = jnp.where(in_win, rel, iv)
            bl_v[sl] = srow >> 7
            t_v[sl] = srow & 127
            msk_v[sl] = jnp.where(in_win, 1, 0)

        for c16 in range(chunk // LANES):
            sl = pl.ds(c16 * LANES, LANES)
            bv = bl_v[sl]
            mv = msk_v[sl]
            tv = t_v[sl]
            for bj in range(LANES // BATCH):
                copies = []
                for j in range(BATCH):
                    lane = bj * BATCH + j
                    bl = bv[lane]
                    m = mv[lane]

                    @pl.when(m == 1)
                    def _(bl=bl, j=j):
                        pltpu.async_copy(trt_hbm.at[bl], slabs_v.at[j], sem0)

                    @pl.when(m == 0)
                    def _(bl=bl, j=j):
                        cb = pl.multiple_of(bl * CB, CB)
                        pltpu.async_copy(
                            memt_hbm.at[:, pl.ds(cb, CB)], slabs_v.at[j], sem0)
                for j in range(BATCH):
                    pltpu.make_async_copy(trt_hbm.at[0], slabs_v.at[j],
                                          sem0).wait()
                for j in range(BATCH):
                    lane = bj * BATCH + j
                    i_local = c16 * LANES + lane
                    tcol = jnp.full((LANES,), tv[lane], jnp.int32)
                    icol = jnp.full((LANES,), i_local, jnp.int32)
                    jv = jnp.full((LANES,), j, jnp.int32)
                    for k in range(D // LANES):
                        dv = k * LANES + lax.iota(jnp.int32, LANES)
                        vals = plsc.load_gather(slabs_v, [jv, dv, tcol])
                        plsc.store_scatter(out_v, [dv, icol], vals)

        pltpu.sync_copy(out_v, out_hbm.at[:, pl.ds(base, chunk)])

    return sc_kernel


def kernel(mem, transition, insert_position, sample_idx):
    data_len = transition.shape[0] * transition.shape[1]
    mem_t = mem.T
    tr_t = jnp.transpose(transition, (0, 2, 1))
    info = plsc.get_sparse_core_info()
    num_workers = info.num_cores * info.num_subcores
    ip_arr = jnp.full((LANES,), insert_position, dtype=jnp.int32)
    sc = _make_sc_kernel(data_len, num_workers)
    out_t = sc(mem_t, tr_t, ip_arr, sample_idx.astype(jnp.int32))
    return out_t.T


# speedup vs baseline: 4.8430x; 1.2888x over previous
import functools

import jax
import jax.numpy as jnp
from jax import lax
from jax.experimental import pallas as pl
from jax.experimental.pallas import tpu as pltpu
from jax.experimental.pallas import tpu_sc as plsc

MAX_SIZE = 1000000
D = 64
SAMPLE_BS = 4096
LANES = 16
CB = 128
SLOTS = 6


def _make_sc_kernel(data_len: int, num_workers: int):
    chunk = SAMPLE_BS // num_workers
    mesh = plsc.VectorSubcoreMesh(core_axis_name="c", subcore_axis_name="s")
    num_cores = mesh.num_cores

    @functools.partial(
        pl.kernel,
        mesh=mesh,
        out_type=jax.ShapeDtypeStruct((D, SAMPLE_BS), jnp.float32),
        compiler_params=pltpu.CompilerParams(needs_layout_passes=False),
        scratch_types=[
            pltpu.VMEM((chunk,), jnp.int32),
            pltpu.VMEM((chunk,), jnp.int32),
            pltpu.VMEM((chunk,), jnp.int32),
            pltpu.VMEM((LANES,), jnp.int32),
            pltpu.VMEM((SLOTS, D, CB), jnp.float32),
            pltpu.VMEM((D, chunk), jnp.float32),
        ] + [pltpu.SemaphoreType.DMA] * SLOTS,
    )
    def sc_kernel(memt_hbm, trt_hbm, ip_hbm, sidx_hbm, out_hbm,
                  msk_v, bl_v, t_v, ip_v, slabs_v, out_v, *sems):
        wid = lax.axis_index("s") * num_cores + lax.axis_index("c")
        base = pl.multiple_of(wid * chunk, chunk)

        pltpu.sync_copy(sidx_hbm.at[pl.ds(base, chunk)], msk_v)
        pltpu.sync_copy(ip_hbm, ip_v)
        ip = ip_v[...]

        for c in range(chunk // LANES):
            sl = pl.ds(c * LANES, LANES)
            iv = msk_v[sl]
            d = iv - ip
            rel = d + jnp.where(d < 0, MAX_SIZE, 0)
            in_win = rel < data_len
            srow = jnp.where(in_win, rel, iv)
            bl_v[sl] = srow >> 7
            t_v[sl] = srow & 127
            msk_v[sl] = jnp.where(in_win, 1, 0)

        def issue(slot, bl, m):
            @pl.when(m == 1)
            def _():
                pltpu.async_copy(trt_hbm.at[bl], slabs_v.at[slot], sems[slot])

            @pl.when(m == 0)
            def _():
                cb = pl.multiple_of(bl * CB, CB)
                pltpu.async_copy(
                    memt_hbm.at[:, pl.ds(cb, CB)], slabs_v.at[slot], sems[slot])

        def drain_extract(slot, i_local, t_s):
            pltpu.make_async_copy(trt_hbm.at[0], slabs_v.at[slot],
                                  sems[slot]).wait()
            tcol = jnp.full((LANES,), t_s, jnp.int32)
            icol = jnp.full((LANES,), i_local, jnp.int32)
            jv = jnp.full((LANES,), slot, jnp.int32)
            for k in range(D // LANES):
                dv = k * LANES + lax.iota(jnp.int32, LANES)
                vals = plsc.load_gather(slabs_v, [jv, dv, tcol])
                plsc.store_scatter(out_v, [dv, icol], vals)

        pending = []
        for c16 in range(chunk // LANES):
            sl = pl.ds(c16 * LANES, LANES)
            bv = bl_v[sl]
            mv = msk_v[sl]
            tv = t_v[sl]
            for lane in range(LANES):
                i = c16 * LANES + lane
                slot = i % SLOTS
                if i >= SLOTS:
                    drain_extract(slot, i - SLOTS, pending[i - SLOTS])
                issue(slot, bv[lane], mv[lane])
                pending.append(tv[lane])
        for i in range(chunk - SLOTS, chunk):
            drain_extract(i % SLOTS, i, pending[i])

        pltpu.sync_copy(out_v, out_hbm.at[:, pl.ds(base, chunk)])

    return sc_kernel


def kernel(mem, transition, insert_position, sample_idx):
    data_len = transition.shape[0] * transition.shape[1]
    mem_t = mem.T
    tr_t = jnp.transpose(transition, (0, 2, 1))
    info = plsc.get_sparse_core_info()
    num_workers = info.num_cores * info.num_subcores
    ip_arr = jnp.full((LANES,), insert_position, dtype=jnp.int32)
    sc = _make_sc_kernel(data_len, num_workers)
    out_t = sc(mem_t, tr_t, ip_arr, sample_idx.astype(jnp.int32))
    return out_t.T
